# baseline (speedup 1.0000x reference)
import functools

import jax
import jax.numpy as jnp
from jax import lax
from jax.experimental import pallas as pl
from jax.experimental.pallas import tpu as pltpu
from jax.experimental.pallas import tpu_sc as plsc

N_EMBED = 1000000
HDIM = 64
HPAD = 128
BATCH = 16384
FIELDS = 26
FPAD = 32
N_TOTAL = BATCH * FIELDS

NW = 32
B_PER_W = N_TOTAL // NW
CHUNK = 128
N_CHUNKS = B_PER_W // CHUNK

NBLK = N_EMBED // HPAD
N_FULL = NBLK * HPAD
BLK_ITERS = (NBLK + NW - 1) // NW

_mesh = plsc.VectorSubcoreMesh(core_axis_name="c", subcore_axis_name="s")


SKEW = HPAD + 1


def _transpose_tile(src, skew, dst, n_cols, iotas):
    def jbody(j8, carry):
        j0 = j8 * 8
        for dj in range(8):
            for k in range(8):
                skew[j0 + dj, pl.ds(16 * k, 16)] = src[j0 + dj, pl.ds(16 * k, 16)]
        return carry

    lax.fori_loop(0, HDIM // 8, jbody, 0)

    def sbody(s16, carry):
        s0 = s16 * 16
        loaded = []
        for u in range(16):
            cs = jnp.full((16,), s0 + u, jnp.int32)
            for m in range(HDIM // 16):
                loaded.append((cs, m, plsc.load_gather(skew, [iotas[m], cs])))
        for cs, m, v in loaded:
            plsc.store_scatter(dst, [cs, iotas[m]], v)
        return carry

    lax.fori_loop(0, n_cols // 16, sbody, 0)


@functools.partial(
    pl.kernel,
    mesh=_mesh,
    out_type=jax.ShapeDtypeStruct((N_EMBED, HPAD), jnp.float32),
    scratch_types=[
        pltpu.VMEM((HDIM, HPAD), jnp.float32),
        pltpu.VMEM((HDIM, HPAD), jnp.float32),
        pltpu.VMEM((HDIM, SKEW), jnp.float32),
        pltpu.VMEM((HPAD, HPAD), jnp.float32),
        pltpu.SemaphoreType.DMA,
        pltpu.SemaphoreType.DMA,
    ],
    compiler_params=pltpu.CompilerParams(
        use_tc_tiling_on_sc=True, needs_layout_passes=False
    ),
)
def _transpose_kernel(embs_t_hbm, tail_t_hbm, table_hbm, tb0, tb1, tsk, ob, sem0, sem1):
    wid = lax.axis_index("s") * 2 + lax.axis_index("c")
    iotas = [lax.iota(jnp.int32, 16) + 16 * k for k in range(8)]
    tbs = (tb0, tb1)
    sems = (sem0, sem1)

    n_mine = jnp.where(wid < NBLK - NW * (BLK_ITERS - 1), BLK_ITERS, BLK_ITERS - 1)

    def fetch(t, slot):
        c = (wid + NW * t) * HPAD

        @pl.when(t < n_mine)
        def _():
            pltpu.async_copy(
                embs_t_hbm.at[:, pl.ds(c, HPAD)], tbs[slot], sems[slot]
            )

    def process(t, slot):
        c = (wid + NW * t) * HPAD

        @pl.when(t < n_mine)
        def _():
            pltpu.make_async_copy(
                embs_t_hbm.at[:, pl.ds(0, HPAD)], tbs[slot], sems[slot]
            ).wait()
            _transpose_tile(tbs[slot], tsk, ob, HPAD, iotas)
            pltpu.sync_copy(ob, table_hbm.at[pl.ds(c, HPAD)])

    fetch(0, 0)

    def body(i, carry):
        t0 = i * 2
        fetch(t0 + 1, 1)
        process(t0, 0)
        fetch(t0 + 2, 0)
        process(t0 + 1, 1)
        return carry

    lax.fori_loop(0, (BLK_ITERS + 1) // 2, body, 0)

    @pl.when(wid == 0)
    def _():
        pltpu.sync_copy(tail_t_hbm, tb0)
        _transpose_tile(tb0, tsk, ob, HDIM, iotas)
        pltpu.sync_copy(
            ob.at[pl.ds(0, N_EMBED - N_FULL)], table_hbm.at[pl.ds(N_FULL, N_EMBED - N_FULL)]
        )


NBUF = 4


@functools.partial(
    pl.kernel,
    mesh=_mesh,
    out_type=jax.ShapeDtypeStruct((BATCH * FPAD, HPAD), jnp.float32),
    scratch_types=[
        pltpu.VMEM((N_CHUNKS, CHUNK), jnp.int32),
        pltpu.VMEM((N_CHUNKS, CHUNK), jnp.int32),
        [pltpu.VMEM((CHUNK, HPAD), jnp.float32) for _ in range(NBUF)],
        [pltpu.SemaphoreType.DMA for _ in range(NBUF)],
        [pltpu.SemaphoreType.DMA for _ in range(NBUF)],
    ],
    compiler_params=pltpu.CompilerParams(use_tc_tiling_on_sc=True),
)
def _gather_kernel(
    idx_hbm, oidx_hbm, table_hbm, out_hbm, idx_v, oidx_v, bufs, sems, osems
):
    wid = lax.axis_index("s") * 2 + lax.axis_index("c")
    pltpu.sync_copy(idx_hbm.at[wid], idx_v)
    pltpu.sync_copy(oidx_hbm.at[wid], oidx_v)

    def gather_start(g, slot):
        pltpu.async_copy(table_hbm.at[idx_v.at[g]], bufs[slot], sems[slot])

    def gather_wait(slot):
        pltpu.make_async_copy(
            table_hbm.at[idx_v.at[0]], bufs[slot], sems[slot]
        ).wait()

    def scatter_start(g, slot):
        pltpu.async_copy(bufs[slot], out_hbm.at[oidx_v.at[g]], osems[slot])

    def scatter_wait(slot):
        pltpu.make_async_copy(
            bufs[slot], out_hbm.at[oidx_v.at[0]], osems[slot]
        ).wait()

    for p in range(NBUF - 1):
        gather_start(p, p)

    def step(g, q):
        gather_wait(q)
        scatter_start(g, q)
        nxt = (q + NBUF - 1) % NBUF

        @pl.when(g >= 1)
        def _():
            scatter_wait(nxt)

        @pl.when(g + NBUF - 1 < N_CHUNKS)
        def _():
            gather_start(g + NBUF - 1, nxt)

    def body(i, carry):
        for q in range(NBUF):
            step(i * NBUF + q, q)
        return carry

    lax.fori_loop(0, N_CHUNKS // NBUF, body, 0)
    scatter_wait((N_CHUNKS - 1) % NBUF)


def _tc_t_body(x_ref, o_ref):
    o_ref[:, :HDIM] = x_ref[...].T


def _tc_transpose(embs_t):
    return pl.pallas_call(
        _tc_t_body,
        out_shape=jax.ShapeDtypeStruct((N_EMBED, HPAD), jnp.float32),
        grid=(NBLK + 1,),
        in_specs=[pl.BlockSpec((HDIM, HPAD), lambda i: (0, i))],
        out_specs=pl.BlockSpec((HPAD, HPAD), lambda i: (i, 0)),
    )(embs_t)


def kernel(indices, embs):
    idx3 = indices.astype(jnp.int32).reshape(NW, N_CHUNKS, CHUNK)
    n = jnp.arange(N_TOTAL, dtype=jnp.int32)
    oidx = ((n // FIELDS) * FPAD + n % FIELDS).reshape(NW, N_CHUNKS, CHUNK)
    table = _tc_transpose(embs.T)
    out = _gather_kernel(idx3, oidx, table)
    return out.reshape(BATCH, FPAD, HPAD)[:, :FIELDS, :HDIM]


# speedup vs baseline: 5.4919x; 5.4919x over previous
import functools

import jax
import jax.numpy as jnp
from jax import lax
from jax.experimental import pallas as pl
from jax.experimental.pallas import tpu as pltpu
from jax.experimental.pallas import tpu_sc as plsc

N_EMBED = 1000000
HDIM = 64
HPAD = 128
BATCH = 16384
FIELDS = 26
FPAD = 32
N_TOTAL = BATCH * FIELDS

NW = 32
B_PER_W = N_TOTAL // NW
CHUNK = 128
N_CHUNKS = B_PER_W // CHUNK

NBLK = N_EMBED // HPAD
N_FULL = NBLK * HPAD
BLK_ITERS = (NBLK + NW - 1) // NW

_mesh = plsc.VectorSubcoreMesh(core_axis_name="c", subcore_axis_name="s")


SKEW = HPAD + 1


def _transpose_tile(src, skew, dst, n_cols, iotas):
    def jbody(j8, carry):
        j0 = j8 * 8
        for dj in range(8):
            for k in range(8):
                skew[j0 + dj, pl.ds(16 * k, 16)] = src[j0 + dj, pl.ds(16 * k, 16)]
        return carry

    lax.fori_loop(0, HDIM // 8, jbody, 0)

    def sbody(s16, carry):
        s0 = s16 * 16
        loaded = []
        for u in range(16):
            cs = jnp.full((16,), s0 + u, jnp.int32)
            for m in range(HDIM // 16):
                loaded.append((cs, m, plsc.load_gather(skew, [iotas[m], cs])))
        for cs, m, v in loaded:
            plsc.store_scatter(dst, [cs, iotas[m]], v)
        return carry

    lax.fori_loop(0, n_cols // 16, sbody, 0)


@functools.partial(
    pl.kernel,
    mesh=_mesh,
    out_type=jax.ShapeDtypeStruct((N_EMBED, HPAD), jnp.float32),
    scratch_types=[
        pltpu.VMEM((HDIM, HPAD), jnp.float32),
        pltpu.VMEM((HDIM, HPAD), jnp.float32),
        pltpu.VMEM((HDIM, SKEW), jnp.float32),
        pltpu.VMEM((HPAD, HPAD), jnp.float32),
        pltpu.SemaphoreType.DMA,
        pltpu.SemaphoreType.DMA,
    ],
    compiler_params=pltpu.CompilerParams(
        use_tc_tiling_on_sc=True, needs_layout_passes=False
    ),
)
def _transpose_kernel(embs_t_hbm, tail_t_hbm, table_hbm, tb0, tb1, tsk, ob, sem0, sem1):
    wid = lax.axis_index("s") * 2 + lax.axis_index("c")
    iotas = [lax.iota(jnp.int32, 16) + 16 * k for k in range(8)]
    tbs = (tb0, tb1)
    sems = (sem0, sem1)

    n_mine = jnp.where(wid < NBLK - NW * (BLK_ITERS - 1), BLK_ITERS, BLK_ITERS - 1)

    def fetch(t, slot):
        c = (wid + NW * t) * HPAD

        @pl.when(t < n_mine)
        def _():
            pltpu.async_copy(
                embs_t_hbm.at[:, pl.ds(c, HPAD)], tbs[slot], sems[slot]
            )

    def process(t, slot):
        c = (wid + NW * t) * HPAD

        @pl.when(t < n_mine)
        def _():
            pltpu.make_async_copy(
                embs_t_hbm.at[:, pl.ds(0, HPAD)], tbs[slot], sems[slot]
            ).wait()
            _transpose_tile(tbs[slot], tsk, ob, HPAD, iotas)
            pltpu.sync_copy(ob, table_hbm.at[pl.ds(c, HPAD)])

    fetch(0, 0)

    def body(i, carry):
        t0 = i * 2
        fetch(t0 + 1, 1)
        process(t0, 0)
        fetch(t0 + 2, 0)
        process(t0 + 1, 1)
        return carry

    lax.fori_loop(0, (BLK_ITERS + 1) // 2, body, 0)

    @pl.when(wid == 0)
    def _():
        pltpu.sync_copy(tail_t_hbm, tb0)
        _transpose_tile(tb0, tsk, ob, HDIM, iotas)
        pltpu.sync_copy(
            ob.at[pl.ds(0, N_EMBED - N_FULL)], table_hbm.at[pl.ds(N_FULL, N_EMBED - N_FULL)]
        )


NBUF = 4


@functools.partial(
    pl.kernel,
    mesh=_mesh,
    out_type=jax.ShapeDtypeStruct((BATCH * FPAD, HPAD), jnp.float32),
    scratch_types=[
        pltpu.VMEM((N_CHUNKS, CHUNK), jnp.int32),
        pltpu.VMEM((N_CHUNKS, CHUNK), jnp.int32),
        [pltpu.VMEM((CHUNK, HPAD), jnp.float32) for _ in range(NBUF)],
        [pltpu.SemaphoreType.DMA for _ in range(NBUF)],
        [pltpu.SemaphoreType.DMA for _ in range(NBUF)],
    ],
    compiler_params=pltpu.CompilerParams(use_tc_tiling_on_sc=True),
)
def _gather_kernel(
    idx_hbm, oidx_hbm, table_hbm, out_hbm, idx_v, oidx_v, bufs, sems, osems
):
    wid = lax.axis_index("s") * 2 + lax.axis_index("c")
    pltpu.sync_copy(idx_hbm.at[wid], idx_v)
    pltpu.sync_copy(oidx_hbm.at[wid], oidx_v)

    def gather_start(g, slot):
        pltpu.async_copy(table_hbm.at[idx_v.at[g]], bufs[slot], sems[slot])

    def gather_wait(slot):
        pltpu.make_async_copy(
            table_hbm.at[idx_v.at[0]], bufs[slot], sems[slot]
        ).wait()

    def scatter_start(g, slot):
        pltpu.async_copy(bufs[slot], out_hbm.at[oidx_v.at[g]], osems[slot])

    def scatter_wait(slot):
        pltpu.make_async_copy(
            bufs[slot], out_hbm.at[oidx_v.at[0]], osems[slot]
        ).wait()

    for p in range(NBUF - 1):
        gather_start(p, p)

    def step(g, q):
        gather_wait(q)
        scatter_start(g, q)
        nxt = (q + NBUF - 1) % NBUF

        @pl.when(g >= 1)
        def _():
            scatter_wait(nxt)

        @pl.when(g + NBUF - 1 < N_CHUNKS)
        def _():
            gather_start(g + NBUF - 1, nxt)

    def body(i, carry):
        for q in range(NBUF):
            step(i * NBUF + q, q)
        return carry

    lax.fori_loop(0, N_CHUNKS // NBUF, body, 0)
    scatter_wait((N_CHUNKS - 1) % NBUF)


def _tc_t_body(x_ref, o_ref):
    o_ref[:, :HDIM] = x_ref[...].T


def _tc_transpose(embs_t):
    return pl.pallas_call(
        _tc_t_body,
        out_shape=jax.ShapeDtypeStruct((N_EMBED, HPAD), jnp.float32),
        grid=(NBLK + 1,),
        in_specs=[pl.BlockSpec((HDIM, HPAD), lambda i: (0, i))],
        out_specs=pl.BlockSpec((HPAD, HPAD), lambda i: (i, 0)),
    )(embs_t)


def kernel(indices, embs):
    idx3 = indices.astype(jnp.int32).reshape(NW, N_CHUNKS, CHUNK)
    n = jnp.arange(N_TOTAL, dtype=jnp.int32)
    oidx = ((n // FIELDS) * FPAD + n % FIELDS).reshape(NW, N_CHUNKS, CHUNK)
    table = jnp.pad(embs, ((0, 0), (0, HPAD - HDIM)))
    out = _gather_kernel(idx3, oidx, table)
    return out.reshape(BATCH, FPAD, HPAD)[:, :FIELDS, :HDIM]
